# baseline (speedup 1.0000x reference)
import functools

import jax
import jax.numpy as jnp
from jax import lax
from jax.experimental import pallas as pl
from jax.experimental.pallas import tpu as pltpu
from jax.experimental.pallas import tpu_sc as plsc

N_NODES = 10000
IN_CH = 128
HID = 128
OUT_CH = 64

NC, NS = 2, 16
NW = NC * NS
B = 128
ACC_ROWS = 10240
RPT = ACC_ROWS // NS
TRASH = ACC_ROWS - 8
ROWS_TC = 1000
DEG_W = 128


def _sc_deg(dst3, ones_h, z_h):
    nb = dst3.shape[1]
    mesh = plsc.VectorSubcoreMesh(core_axis_name="c", subcore_axis_name="s")

    @functools.partial(
        pl.kernel,
        out_type=jax.ShapeDtypeStruct((NC, ACC_ROWS, DEG_W), jnp.float32),
        mesh=mesh,
        scratch_types=[
            pltpu.VMEM((nb, B), jnp.int32),
            pltpu.VMEM((B, DEG_W), jnp.float32),
            pltpu.VMEM_SHARED((ACC_ROWS, DEG_W), jnp.float32),
        ],
    )
    def run(dst_h, ones_hbm, z_hbm, degp, idx_v, ones_v, acc):
        c = lax.axis_index("c")
        s = lax.axis_index("s")
        w = s * NC + c
        base = s * RPT
        pltpu.sync_copy(z_hbm, acc.at[pl.ds(base, RPT)])
        pltpu.sync_copy(ones_hbm, ones_v)
        pltpu.sync_copy(dst_h.at[w], idx_v)
        plsc.subcore_barrier()

        def body(j, carry):
            pltpu.sync_copy(ones_v, acc.at[idx_v.at[j]], add=True)
            return carry

        lax.fori_loop(0, nb, body, 0)
        plsc.subcore_barrier()
        pltpu.sync_copy(acc.at[pl.ds(base, RPT)], degp.at[c, pl.ds(base, RPT)])

    return run(dst3, ones_h, z_h)


def _sc_agg(g, src3, dst3, z_h, d):
    nb = src3.shape[1]
    mesh = plsc.VectorSubcoreMesh(core_axis_name="c", subcore_axis_name="s")

    @functools.partial(
        pl.kernel,
        out_type=jax.ShapeDtypeStruct((NC, ACC_ROWS, d), jnp.float32),
        mesh=mesh,
        scratch_types=[
            pltpu.VMEM((nb, B), jnp.int32),
            pltpu.VMEM((nb, B), jnp.int32),
            pltpu.VMEM((B, d), jnp.float32),
            pltpu.VMEM_SHARED((ACC_ROWS, d), jnp.float32),
        ],
    )
    def run(g_h, src_h, dst_h, z_hbm, part, src_v, dst_v, rows_v, acc):
        c = lax.axis_index("c")
        s = lax.axis_index("s")
        w = s * NC + c
        base = s * RPT
        pltpu.sync_copy(z_hbm, acc.at[pl.ds(base, RPT)])
        pltpu.sync_copy(src_h.at[w], src_v)
        pltpu.sync_copy(dst_h.at[w], dst_v)
        plsc.subcore_barrier()

        def body(j, carry):
            pltpu.sync_copy(g_h.at[src_v.at[j]], rows_v)
            pltpu.sync_copy(rows_v, acc.at[dst_v.at[j]], add=True)
            return carry

        lax.fori_loop(0, nb, body, 0)
        plsc.subcore_barrier()
        pltpu.sync_copy(acc.at[pl.ds(base, RPT)], part.at[c, pl.ds(base, RPT)])

    return run(g, src3, dst3, z_h)


def _dis_block(d_ref):
    return lax.rsqrt(d_ref[...] + 1.0)


def _tc_a_body(x_ref, w_ref, d_ref, o_ref):
    dis = _dis_block(d_ref)
    h = jnp.dot(x_ref[...], w_ref[...], preferred_element_type=jnp.float32)
    o_ref[...] = h * dis


def _tc_a(x, W1, degp):
    return pl.pallas_call(
        _tc_a_body,
        grid=(N_NODES // ROWS_TC,),
        in_specs=[
            pl.BlockSpec((ROWS_TC, IN_CH), lambda i: (i, 0)),
            pl.BlockSpec((IN_CH, HID), lambda i: (0, 0)),
            pl.BlockSpec((ROWS_TC, 1), lambda i: (i, 0)),
        ],
        out_specs=pl.BlockSpec((ROWS_TC, HID), lambda i: (i, 0)),
        out_shape=jax.ShapeDtypeStruct((N_NODES, HID), jnp.float32),
    )(x, W1, degp)


def _tc_b_body(p_ref, g1_ref, d_ref, b1_ref, w2_ref, o_ref):
    dis = _dis_block(d_ref)
    agg = p_ref[0] + p_ref[1] + g1_ref[...]
    y = jnp.maximum(agg * dis + b1_ref[...], 0.0)
    h2 = jnp.dot(y, w2_ref[...], preferred_element_type=jnp.float32)
    o_ref[...] = jnp.concatenate(
        [h2 * dis, jnp.zeros((ROWS_TC, HID - OUT_CH), jnp.float32)], axis=1
    )


def _tc_b(p1, g1, degp, b1, W2):
    return pl.pallas_call(
        _tc_b_body,
        grid=(N_NODES // ROWS_TC,),
        in_specs=[
            pl.BlockSpec((2, ROWS_TC, HID), lambda i: (0, i, 0)),
            pl.BlockSpec((ROWS_TC, HID), lambda i: (i, 0)),
            pl.BlockSpec((ROWS_TC, 1), lambda i: (i, 0)),
            pl.BlockSpec((1, HID), lambda i: (0, 0)),
            pl.BlockSpec((HID, OUT_CH), lambda i: (0, 0)),
        ],
        out_specs=pl.BlockSpec((ROWS_TC, HID), lambda i: (i, 0)),
        out_shape=jax.ShapeDtypeStruct((N_NODES, HID), jnp.float32),
    )(p1, g1, degp, b1, W2)


def _tc_c_body(q_ref, g2_ref, d_ref, b2_ref, o_ref):
    dis = _dis_block(d_ref)
    agg = q_ref[0, :, 0:OUT_CH] + q_ref[1, :, 0:OUT_CH] + g2_ref[:, 0:OUT_CH]
    o_ref[...] = agg * dis + b2_ref[...]


def _tc_c(p2, g2, degp, b2):
    return pl.pallas_call(
        _tc_c_body,
        grid=(N_NODES // ROWS_TC,),
        in_specs=[
            pl.BlockSpec((2, ROWS_TC, HID), lambda i: (0, i, 0)),
            pl.BlockSpec((ROWS_TC, HID), lambda i: (i, 0)),
            pl.BlockSpec((ROWS_TC, 1), lambda i: (i, 0)),
            pl.BlockSpec((1, OUT_CH), lambda i: (0, 0)),
        ],
        out_specs=pl.BlockSpec((ROWS_TC, OUT_CH), lambda i: (i, 0)),
        out_shape=jax.ShapeDtypeStruct((N_NODES, OUT_CH), jnp.float32),
    )(p2, g2, degp, b2)


def kernel(x, edge_index, W1, b1, W2, b2):
    src = edge_index[0].astype(jnp.int32)
    dst = edge_index[1].astype(jnp.int32)
    e = src.shape[0]
    nb = -(-e // (NW * B))
    ep = NW * B * nb
    pad = ep - e
    src3 = jnp.concatenate([src, jnp.zeros((pad,), jnp.int32)]).reshape(NW, nb, B)
    dst3 = jnp.concatenate([dst, jnp.full((pad,), TRASH, jnp.int32)]).reshape(NW, nb, B)
    z128 = jnp.zeros((RPT, HID), jnp.float32)
    zdeg = jnp.zeros((RPT, DEG_W), jnp.float32)
    ones_h = jnp.ones((B, DEG_W), jnp.float32)

    degp = _sc_deg(dst3, ones_h, zdeg)
    deg = (degp[0, :N_NODES, 0] + degp[1, :N_NODES, 0])[:, None]
    g1 = _tc_a(x, W1, deg)
    p1 = _sc_agg(g1, src3, dst3, z128, HID)
    g2 = _tc_b(p1, g1, deg, b1.reshape(1, HID), W2)
    p2 = _sc_agg(g2, src3, dst3, z128, HID)
    out = _tc_c(p2, g2, deg, b2.reshape(1, OUT_CH))
    return out


# speedup vs baseline: 1.5930x; 1.5930x over previous
import functools

import jax
import jax.numpy as jnp
from jax import lax
from jax.experimental import pallas as pl
from jax.experimental.pallas import tpu as pltpu
from jax.experimental.pallas import tpu_sc as plsc

N_NODES = 10000
IN_CH = 128
HID = 128
OUT_CH = 64

NC, NS = 2, 16
NW = NC * NS
B = 128
ACC_ROWS = 10240
RPT = ACC_ROWS // NS
TRASH = ACC_ROWS - 8
ROWS_TC = 1000
DEG_W = 128


def _sc_deg(dst3, ones_h, z_h):
    nb = dst3.shape[1]
    mesh = plsc.VectorSubcoreMesh(core_axis_name="c", subcore_axis_name="s")

    @functools.partial(
        pl.kernel,
        out_type=jax.ShapeDtypeStruct((NC, ACC_ROWS, DEG_W), jnp.float32),
        mesh=mesh,
        scratch_types=[
            pltpu.VMEM((nb, B), jnp.int32),
            pltpu.VMEM((B, DEG_W), jnp.float32),
            pltpu.VMEM_SHARED((ACC_ROWS, DEG_W), jnp.float32),
        ],
    )
    def run(dst_h, ones_hbm, z_hbm, degp, idx_v, ones_v, acc):
        c = lax.axis_index("c")
        s = lax.axis_index("s")
        w = s * NC + c
        base = s * RPT
        pltpu.sync_copy(z_hbm, acc.at[pl.ds(base, RPT)])
        pltpu.sync_copy(ones_hbm, ones_v)
        pltpu.sync_copy(dst_h.at[w], idx_v)
        plsc.subcore_barrier()

        def body(j, carry):
            pltpu.sync_copy(ones_v, acc.at[idx_v.at[j]], add=True)
            return carry

        lax.fori_loop(0, nb, body, 0)
        plsc.subcore_barrier()
        pltpu.sync_copy(acc.at[pl.ds(base, RPT)], degp.at[c, pl.ds(base, RPT)])

    return run(dst3, ones_h, z_h)


def _sc_agg(g, src3, dst3, z_h, d):
    nb = src3.shape[1]
    mesh = plsc.VectorSubcoreMesh(core_axis_name="c", subcore_axis_name="s")

    @functools.partial(
        pl.kernel,
        out_type=jax.ShapeDtypeStruct((NC, ACC_ROWS, d), jnp.float32),
        mesh=mesh,
        scratch_types=[
            pltpu.VMEM((nb, B), jnp.int32),
            pltpu.VMEM((nb, B), jnp.int32),
            pltpu.VMEM((B, d), jnp.float32),
            pltpu.VMEM_SHARED((ACC_ROWS, d), jnp.float32),
        ],
    )
    def run(g_h, src_h, dst_h, z_hbm, part, src_v, dst_v, rows_v, acc):
        c = lax.axis_index("c")
        s = lax.axis_index("s")
        w = s * NC + c
        base = s * RPT
        pltpu.sync_copy(z_hbm, acc.at[pl.ds(base, RPT)])
        pltpu.sync_copy(src_h.at[w], src_v)
        pltpu.sync_copy(dst_h.at[w], dst_v)
        plsc.subcore_barrier()

        def body(j, carry):
            pltpu.sync_copy(g_h.at[src_v.at[j]], rows_v)
            pltpu.sync_copy(rows_v, acc.at[dst_v.at[j]], add=True)
            return carry

        lax.fori_loop(0, nb, body, 0)
        plsc.subcore_barrier()
        pltpu.sync_copy(acc.at[pl.ds(base, RPT)], part.at[c, pl.ds(base, RPT)])

    return run(g, src3, dst3, z_h)


def _dis_block(d_ref):
    return lax.rsqrt(d_ref[...] + 1.0)


def _tc_a_body(x_ref, w_ref, d_ref, o_ref):
    dis = _dis_block(d_ref)
    h = jnp.dot(x_ref[...], w_ref[...], preferred_element_type=jnp.float32)
    o_ref[...] = h * dis


def _tc_a(x, W1, degp):
    return pl.pallas_call(
        _tc_a_body,
        grid=(N_NODES // ROWS_TC,),
        in_specs=[
            pl.BlockSpec((ROWS_TC, IN_CH), lambda i: (i, 0)),
            pl.BlockSpec((IN_CH, HID), lambda i: (0, 0)),
            pl.BlockSpec((ROWS_TC, 1), lambda i: (i, 0)),
        ],
        out_specs=pl.BlockSpec((ROWS_TC, HID), lambda i: (i, 0)),
        out_shape=jax.ShapeDtypeStruct((N_NODES, HID), jnp.float32),
    )(x, W1, degp)


def _tc_b_body(p_ref, g1_ref, d_ref, b1_ref, w2_ref, o_ref):
    dis = _dis_block(d_ref)
    agg = p_ref[0] + p_ref[1] + g1_ref[...]
    y = jnp.maximum(agg * dis + b1_ref[...], 0.0)
    h2 = jnp.dot(y, w2_ref[...], preferred_element_type=jnp.float32)
    o_ref[...] = jnp.concatenate(
        [h2 * dis, jnp.zeros((ROWS_TC, HID - OUT_CH), jnp.float32)], axis=1
    )


def _tc_b(p1, g1, degp, b1, W2):
    return pl.pallas_call(
        _tc_b_body,
        grid=(N_NODES // ROWS_TC,),
        in_specs=[
            pl.BlockSpec((2, ROWS_TC, HID), lambda i: (0, i, 0)),
            pl.BlockSpec((ROWS_TC, HID), lambda i: (i, 0)),
            pl.BlockSpec((ROWS_TC, 1), lambda i: (i, 0)),
            pl.BlockSpec((1, HID), lambda i: (0, 0)),
            pl.BlockSpec((HID, OUT_CH), lambda i: (0, 0)),
        ],
        out_specs=pl.BlockSpec((ROWS_TC, HID), lambda i: (i, 0)),
        out_shape=jax.ShapeDtypeStruct((N_NODES, HID), jnp.float32),
    )(p1, g1, degp, b1, W2)


def _tc_c_body(q_ref, g2_ref, d_ref, b2_ref, o_ref):
    dis = _dis_block(d_ref)
    agg = q_ref[0, :, 0:OUT_CH] + q_ref[1, :, 0:OUT_CH] + g2_ref[:, 0:OUT_CH]
    o_ref[...] = agg * dis + b2_ref[...]


def _tc_c(p2, g2, degp, b2):
    return pl.pallas_call(
        _tc_c_body,
        grid=(N_NODES // ROWS_TC,),
        in_specs=[
            pl.BlockSpec((2, ROWS_TC, HID), lambda i: (0, i, 0)),
            pl.BlockSpec((ROWS_TC, HID), lambda i: (i, 0)),
            pl.BlockSpec((ROWS_TC, 1), lambda i: (i, 0)),
            pl.BlockSpec((1, OUT_CH), lambda i: (0, 0)),
        ],
        out_specs=pl.BlockSpec((ROWS_TC, OUT_CH), lambda i: (i, 0)),
        out_shape=jax.ShapeDtypeStruct((N_NODES, OUT_CH), jnp.float32),
    )(p2, g2, degp, b2)


def kernel(x, edge_index, W1, b1, W2, b2):
    src = edge_index[0].astype(jnp.int32)
    dst = edge_index[1].astype(jnp.int32)
    e = src.shape[0]
    nb = -(-e // (NW * B))
    ep = NW * B * nb
    pad = ep - e
    pad_dst = N_NODES + (jnp.arange(pad, dtype=jnp.int32) % (ACC_ROWS - N_NODES))
    pad_src = jnp.arange(pad, dtype=jnp.int32) % N_NODES
    src3 = jnp.concatenate([src, pad_src]).reshape(NW, nb, B)
    dst3 = jnp.concatenate([dst, pad_dst]).reshape(NW, nb, B)
    z128 = jnp.zeros((RPT, HID), jnp.float32)
    zdeg = jnp.zeros((RPT, DEG_W), jnp.float32)
    ones_h = jnp.ones((B, DEG_W), jnp.float32)

    degp = _sc_deg(dst3, ones_h, zdeg)
    deg = (degp[0, :N_NODES, 0] + degp[1, :N_NODES, 0])[:, None]
    g1 = _tc_a(x, W1, deg)
    p1 = _sc_agg(g1, src3, dst3, z128, HID)
    g2 = _tc_b(p1, g1, deg, b1.reshape(1, HID), W2)
    p2 = _sc_agg(g2, src3, dst3, z128, HID)
    out = _tc_c(p2, g2, deg, b2.reshape(1, OUT_CH))
    return out
